# baseline (speedup 1.0000x reference)
import functools

import jax
import jax.numpy as jnp
from jax import lax
from jax.experimental import pallas as pl
from jax.experimental.pallas import tpu as pltpu
from jax.experimental.pallas import tpu_sc as plsc

NC = 2
NS = 16
L = 16
EMBED = 128
BLK = 64
NBUF = 4


def _fuse_tables_body(typ_ref, pos_ref, out_ref):
    p = pos_ref[...]
    t = typ_ref[...]
    out_ref[...] = t[:, None, :] + p[None, :, :]


def _fuse_tables(type_emb, pos_emb):
    tv, e = type_emb.shape
    mp, _ = pos_emb.shape
    out = pl.pallas_call(
        _fuse_tables_body,
        out_shape=jax.ShapeDtypeStruct((tv, mp, e), jnp.float32),
    )(type_emb, pos_emb)
    rows = tv * mp
    packed = lax.bitcast_convert_type(
        out.reshape(rows, e // 2, 2).astype(jnp.bfloat16), jnp.int32)
    return packed.reshape(rows * (e // 2))


def _sc_body(nblk, ids_hbm, pid_hbm, tt_hbm, word_hbm, fused_hbm,
             out_hbm, ids_v, pid_v, tt_v, fidx_v, wbuf, fused_vm, sem_i,
             sem_w0, sem_w1, sem_w2, sem_w3,
             sem_o0, sem_o1, sem_o2, sem_o3):
    sems_w = (sem_w0, sem_w1, sem_w2, sem_w3)
    sems_o = (sem_o0, sem_o1, sem_o2, sem_o3)
    max_pos = fused_hbm.shape[0] // EMBED
    wid = lax.axis_index("s") * NC + lax.axis_index("c")
    base = wid * (nblk * BLK)

    pltpu.sync_copy(fused_hbm, fused_vm)

    def issue(g, s):
        t0 = base + g * BLK
        c1 = pltpu.async_copy(ids_hbm.at[pl.ds(t0, BLK)], ids_v.at[s], sem_i)
        c2 = pltpu.async_copy(pid_hbm.at[pl.ds(t0, BLK)], pid_v.at[s], sem_i)
        c3 = pltpu.async_copy(tt_hbm.at[pl.ds(t0, BLK)], tt_v.at[s], sem_i)
        c1.wait()
        c2.wait()
        c3.wait()
        for k in range(BLK // L):
            sl = pl.ds(k * L, L)
            fidx_v[s, sl] = tt_v[s, sl] * max_pos + pid_v[s, sl]
        pltpu.async_copy(word_hbm.at[ids_v.at[s]], wbuf.at[s], sems_w[s])

    def wait_gather(s):
        pltpu.make_async_copy(word_hbm.at[ids_v.at[s]], wbuf.at[s],
                              sems_w[s]).wait()

    def wait_out(s):
        pltpu.make_async_copy(wbuf.at[s], out_hbm.at[pl.ds(base, BLK)],
                              sems_o[s]).wait()

    def add_and_store(g, s):
        def ak(jg, c2):
            fv = fidx_v[s, pl.ds(jg * L, L)]
            ridx = fv * (EMBED // 2)
            tok_idx = lax.iota(jnp.int32, L) + jg * L
            for w in range(EMBED // 2):
                v = plsc.load_gather(fused_vm, [ridx + w])
                a = lax.bitcast_convert_type(
                    lax.shift_left(v, 16), jnp.float32)
                b = lax.bitcast_convert_type(
                    lax.bitwise_and(v, jnp.int32(-65536)), jnp.float32)
                ca = jnp.full((L,), 2 * w, jnp.int32)
                cb = jnp.full((L,), 2 * w + 1, jnp.int32)
                plsc.addupdate_scatter(wbuf.at[s], [tok_idx, ca], a)
                plsc.addupdate_scatter(wbuf.at[s], [tok_idx, cb], b)
            return c2

        lax.fori_loop(0, BLK // L, ak, 0)
        t0 = base + g * BLK
        pltpu.async_copy(wbuf.at[s], out_hbm.at[pl.ds(t0, BLK)], sems_o[s])

    nout = nblk // NBUF
    issue(0, 0)

    def outer(g0, carry):
        for b in range(NBUF):
            g = g0 * NBUF + b
            s = b
            ns = (b + 1) % NBUF
            if b < NBUF - 1:
                @pl.when(g0 >= 1)
                def _():
                    wait_out(ns)
                issue(g + 1, ns)
            else:
                @pl.when(g0 < nout - 1)
                def _():
                    wait_out(ns)
                    issue(g + 1, ns)
            wait_gather(s)
            add_and_store(g, s)
        return carry

    lax.fori_loop(0, nout, outer, 0)
    for s in range(NBUF):
        wait_out(s)


def kernel(input_ids, position_ids, token_type_ids, word_embeddings,
           position_embeddings, token_type_embeddings):
    batch, seqlen = input_ids.shape
    tok = batch * seqlen
    nw = NC * NS
    per_w = tok // nw
    nblk = per_w // BLK

    ids = input_ids.reshape(-1).astype(jnp.int32)
    pid = position_ids.reshape(-1).astype(jnp.int32)
    tt = token_type_ids.reshape(-1).astype(jnp.int32)

    fused = _fuse_tables(token_type_embeddings, position_embeddings)

    mesh = plsc.VectorSubcoreMesh(core_axis_name="c", subcore_axis_name="s")
    sc = pl.kernel(
        functools.partial(_sc_body, nblk),
        out_type=jax.ShapeDtypeStruct((tok, EMBED), jnp.float32),
        mesh=mesh,
        compiler_params=pltpu.CompilerParams(needs_layout_passes=False),
        scratch_types=[
            pltpu.VMEM((NBUF, BLK), jnp.int32),
            pltpu.VMEM((NBUF, BLK), jnp.int32),
            pltpu.VMEM((NBUF, BLK), jnp.int32),
            pltpu.VMEM((NBUF, BLK), jnp.int32),
            pltpu.VMEM((NBUF, BLK, EMBED), jnp.float32),
            pltpu.VMEM(fused.shape, jnp.int32),
        ] + [pltpu.SemaphoreType.DMA] * 9,
    )
    out = sc(ids, pid, tt, word_embeddings, fused)
    return out.reshape(batch, seqlen, EMBED)


# speedup vs baseline: 5.2546x; 5.2546x over previous
import functools

import jax
import jax.numpy as jnp
from jax import lax
from jax.experimental import pallas as pl
from jax.experimental.pallas import tpu as pltpu
from jax.experimental.pallas import tpu_sc as plsc

NC = 2
NS = 16
L = 16
EMBED = 128
BLK = 64
NBUF = 4


def _fuse_tables_body(typ_ref, pos_ref, out_ref):
    p = pos_ref[...]
    t = typ_ref[...]
    out_ref[...] = t[:, None, :] + p[None, :, :]


def _fuse_tables(type_emb, pos_emb):
    tv, e = type_emb.shape
    mp, _ = pos_emb.shape
    out = pl.pallas_call(
        _fuse_tables_body,
        out_shape=jax.ShapeDtypeStruct((tv, mp, e), jnp.float32),
    )(type_emb, pos_emb)
    return out.reshape(tv * mp, e)


def _sc_body(nblk, max_pos, ids_hbm, pid_hbm, tt_hbm, word_hbm, fused_hbm,
             out_hbm, ids_v, pid_v, tt_v, fidx_v, wbuf, pbuf, sem_i,
             sem_w0, sem_w1, sem_w2, sem_w3,
             sem_p0, sem_p1, sem_p2, sem_p3,
             sem_o0, sem_o1, sem_o2, sem_o3):
    sems_w = (sem_w0, sem_w1, sem_w2, sem_w3)
    sems_p = (sem_p0, sem_p1, sem_p2, sem_p3)
    sems_o = (sem_o0, sem_o1, sem_o2, sem_o3)
    wid = lax.axis_index("s") * NC + lax.axis_index("c")
    base = wid * (nblk * BLK)

    def issue(g, s):
        t0 = base + g * BLK
        c1 = pltpu.async_copy(ids_hbm.at[pl.ds(t0, BLK)], ids_v.at[s], sem_i)
        c2 = pltpu.async_copy(pid_hbm.at[pl.ds(t0, BLK)], pid_v.at[s], sem_i)
        c3 = pltpu.async_copy(tt_hbm.at[pl.ds(t0, BLK)], tt_v.at[s], sem_i)
        c1.wait()
        c2.wait()
        c3.wait()
        for k in range(BLK // L):
            sl = pl.ds(k * L, L)
            fidx_v[s, sl] = tt_v[s, sl] * max_pos + pid_v[s, sl]
        pltpu.async_copy(word_hbm.at[ids_v.at[s]], wbuf.at[s], sems_w[s])
        pltpu.async_copy(fused_hbm.at[fidx_v.at[s]], pbuf.at[s], sems_p[s])

    def wait_gathers(s):
        pltpu.make_async_copy(word_hbm.at[ids_v.at[s]], wbuf.at[s],
                              sems_w[s]).wait()
        pltpu.make_async_copy(fused_hbm.at[fidx_v.at[s]], pbuf.at[s],
                              sems_p[s]).wait()

    def wait_out(s):
        pltpu.make_async_copy(pbuf.at[s], out_hbm.at[pl.ds(base, BLK)],
                              sems_o[s]).wait()

    def add_and_store(g, s):
        def ak(j, c2):
            for c in range(EMBED // L):
                sl = pl.ds(c * L, L)
                plsc.addupdate(pbuf.at[s, j, sl], wbuf[s, j, sl])
            return c2

        lax.fori_loop(0, BLK, ak, 0)
        t0 = base + g * BLK
        pltpu.async_copy(pbuf.at[s], out_hbm.at[pl.ds(t0, BLK)], sems_o[s])

    nout = nblk // NBUF
    issue(0, 0)

    def outer(g0, carry):
        for b in range(NBUF):
            g = g0 * NBUF + b
            s = b
            ns = (b + 1) % NBUF
            if b < NBUF - 1:
                @pl.when(g0 >= 1)
                def _():
                    wait_out(ns)
                issue(g + 1, ns)
            else:
                @pl.when(g0 < nout - 1)
                def _():
                    wait_out(ns)
                    issue(g + 1, ns)
            wait_gathers(s)
            add_and_store(g, s)
        return carry

    lax.fori_loop(0, nout, outer, 0)
    for s in range(NBUF):
        wait_out(s)


def kernel(input_ids, position_ids, token_type_ids, word_embeddings,
           position_embeddings, token_type_embeddings):
    batch, seqlen = input_ids.shape
    tok = batch * seqlen
    nw = NC * NS
    per_w = tok // nw
    nblk = per_w // BLK
    max_pos = position_embeddings.shape[0]

    ids = input_ids.reshape(-1).astype(jnp.int32)
    pid = position_ids.reshape(-1).astype(jnp.int32)
    tt = token_type_ids.reshape(-1).astype(jnp.int32)

    fused = _fuse_tables(token_type_embeddings, position_embeddings)

    mesh = plsc.VectorSubcoreMesh(core_axis_name="c", subcore_axis_name="s")
    sc = pl.kernel(
        functools.partial(_sc_body, nblk, max_pos),
        out_type=jax.ShapeDtypeStruct((tok, EMBED), jnp.float32),
        mesh=mesh,
        scratch_types=[
            pltpu.VMEM((NBUF, BLK), jnp.int32),
            pltpu.VMEM((NBUF, BLK), jnp.int32),
            pltpu.VMEM((NBUF, BLK), jnp.int32),
            pltpu.VMEM((NBUF, BLK), jnp.int32),
            pltpu.VMEM((NBUF, BLK, EMBED), jnp.float32),
            pltpu.VMEM((NBUF, BLK, EMBED), jnp.float32),
        ] + [pltpu.SemaphoreType.DMA] * 13,
    )
    out = sc(ids, pid, tt, word_embeddings, fused)
    return out.reshape(batch, seqlen, EMBED)
